# baseline (speedup 1.0000x reference)
import jax
import jax.numpy as jnp
from jax import lax
from jax.experimental import pallas as pl
from jax.experimental.pallas import tpu as pltpu
from jax.experimental.pallas import tpu_sc as plsc

B, AT, NBR, NFEAT = 2, 512, 16, 128
NM = NBR - 1
RPA = NBR * NM
NC, NS = 2, 16
NW = NC * NS
NATOMS = B * AT
APW = NATOMS // NW
WPB = AT // APW


def _write_runs(out_hbm, rows_v, wsem, bb, ab, issue):
    copy = pltpu.async_copy if issue else (
        lambda s, d, m: pltpu.make_async_copy(s, d, m).wait()
    )
    for i in range(NBR):
        if i > 0:
            copy(
                rows_v.at[:, pl.ds(0, i)],
                out_hbm.at[bb, pl.ds(ab, APW), i, pl.ds(0, i)],
                wsem,
            )
        if i < NBR - 1:
            copy(
                rows_v.at[:, pl.ds(i + 1, NM - i)],
                out_hbm.at[bb, pl.ds(ab, APW), i, pl.ds(i, NM - i)],
                wsem,
            )


def _sc_body(emb_hbm, nbr_hbm, out_hbm, nbr_v, rows_v, gsem, wsem):
    wid = lax.axis_index("s") * NC + lax.axis_index("c")
    base = wid * APW
    bb = wid // WPB
    ab = (wid % WPB) * APW
    pltpu.sync_copy(nbr_hbm.at[pl.ds(base, APW)], nbr_v)
    for a in range(APW):
        pltpu.async_copy(emb_hbm.at[nbr_v.at[a]], rows_v.at[a], gsem)
    for a in range(APW):
        pltpu.make_async_copy(emb_hbm.at[nbr_v.at[a]], rows_v.at[a], gsem).wait()
    _write_runs(out_hbm, rows_v, wsem, bb, ab, True)
    _write_runs(out_hbm, rows_v, wsem, bb, ab, False)


def kernel(node_embedding, nbr_idx):
    emb_flat = node_embedding.reshape(NATOMS, NFEAT)
    batch_off = (jnp.arange(B, dtype=jnp.int32) * AT)[:, None, None]
    nbr_glob = (nbr_idx.astype(jnp.int32) + batch_off).reshape(NATOMS, NBR)

    run = pl.kernel(
        _sc_body,
        out_type=jax.ShapeDtypeStruct((B, AT, NBR, NM, NFEAT), jnp.float32),
        mesh=plsc.VectorSubcoreMesh(core_axis_name="c", subcore_axis_name="s"),
        scratch_types=[
            pltpu.VMEM((APW, NBR), jnp.int32),
            pltpu.VMEM((APW, NBR, NFEAT), jnp.float32),
            pltpu.SemaphoreType.DMA,
            pltpu.SemaphoreType.DMA,
        ],
        compiler_params=pltpu.CompilerParams(
            needs_layout_passes=False, use_tc_tiling_on_sc=False
        ),
    )
    return run(emb_flat, nbr_glob)


# speedup vs baseline: 1.8112x; 1.8112x over previous
import jax
import jax.numpy as jnp
from jax import lax
from jax.experimental import pallas as pl
from jax.experimental.pallas import tpu as pltpu
from jax.experimental.pallas import tpu_sc as plsc

B, AT, NBR, NFEAT = 2, 512, 16, 128
NM = NBR - 1
RPA = NBR * NM
NC, NS = 2, 16
NW = NC * NS
NATOMS = B * AT
APW = NATOMS // NW
WPB = AT // APW


def _write_runs(out_hbm, rows_v, wsem, bb, ab, issue):
    copy = pltpu.async_copy if issue else (
        lambda s, d, m: pltpu.make_async_copy(s, d, m).wait()
    )
    for i in range(NBR):
        if i > 0:
            copy(
                rows_v.at[:, pl.ds(0, i)],
                out_hbm.at[bb, pl.ds(ab, APW), i, pl.ds(0, i)],
                wsem,
            )
        if i < NBR - 1:
            copy(
                rows_v.at[:, pl.ds(i + 1, NM - i)],
                out_hbm.at[bb, pl.ds(ab, APW), i, pl.ds(i, NM - i)],
                wsem,
            )


def _sc_body(emb_hbm, nbr_hbm, out_hbm, nbr_v, rows_v, gsem, wsem):
    wid = lax.axis_index("s") * NC + lax.axis_index("c")
    base = wid * APW
    bb = wid // WPB
    ab = (wid % WPB) * APW
    pltpu.sync_copy(nbr_hbm.at[pl.ds(base, APW)], nbr_v)
    for a in range(APW):
        pltpu.async_copy(emb_hbm.at[nbr_v.at[a]], rows_v.at[a], gsem)
    for a in range(APW):
        pltpu.make_async_copy(emb_hbm.at[nbr_v.at[a]], rows_v.at[a], gsem).wait()
    _write_runs(out_hbm, rows_v, wsem, bb, ab, True)
    _write_runs(out_hbm, rows_v, wsem, bb, ab, False)


def kernel(node_embedding, nbr_idx):
    emb_flat = node_embedding.reshape(NATOMS, NFEAT)
    batch_off = (jnp.arange(B, dtype=jnp.int32) * AT)[:, None, None]
    nbr_glob = (nbr_idx.astype(jnp.int32) + batch_off).reshape(NATOMS, NBR)

    run = pl.kernel(
        _sc_body,
        out_type=jax.ShapeDtypeStruct((B, AT, NBR, NBR, NFEAT), jnp.float32),
        mesh=plsc.VectorSubcoreMesh(core_axis_name="c", subcore_axis_name="s"),
        scratch_types=[
            pltpu.VMEM((APW, NBR), jnp.int32),
            pltpu.VMEM((APW, NBR, NFEAT), jnp.float32),
            pltpu.SemaphoreType.DMA,
            pltpu.SemaphoreType.DMA,
        ],
        compiler_params=pltpu.CompilerParams(
            needs_layout_passes=False, use_tc_tiling_on_sc=False
        ),
    )
    return run(emb_flat, nbr_glob)[:, :, :, :NM, :]
